# baseline (speedup 1.0000x reference)
import functools

import jax
import jax.numpy as jnp
from jax import lax
from jax.experimental import pallas as pl
from jax.experimental.pallas import tpu as pltpu
from jax.experimental.pallas import tpu_sc as plsc

B = 64
P = 576
D = 768

_NC = 2
_NS = 16
_NW = _NC * _NS
_PPW = P // _NW
_CH = _PPW * D
_GROUPS = _CH // 16


def _sc_body(x_hbm, t_hbm, o_hbm, pos_v, ib0, ib1, ob0, ob1,
             is0, is1, os0, os1):
    cid = lax.axis_index("c")
    sid = lax.axis_index("s")
    wid = sid * _NC + cid

    pltpu.sync_copy(t_hbm.at[pl.ds(wid * _CH, _CH)], pos_v)

    def x_off(b):
        return b * (P * D) + wid * _CH

    ibufs = (ib0, ib1)
    obufs = (ob0, ob1)
    isems = (is0, is1)
    osems = (os0, os1)

    pltpu.make_async_copy(x_hbm.at[pl.ds(x_off(0), _CH)], ib0, is0).start()
    pltpu.make_async_copy(x_hbm.at[pl.ds(x_off(1), _CH)], ib1, is1).start()

    def step(b, ibuf, obuf, isem, osem):
        pltpu.make_async_copy(x_hbm.at[pl.ds(x_off(b), _CH)], ibuf, isem).wait()

        @pl.when(b >= 2)
        def _():
            pltpu.make_async_copy(
                obuf, o_hbm.at[pl.ds(x_off(b - 2), _CH)], osem).wait()

        def add_group(i, carry):
            g = pl.multiple_of(i * 16, 16)
            obuf[pl.ds(g, 16)] = ibuf[pl.ds(g, 16)] + pos_v[pl.ds(g, 16)]
            return carry

        lax.fori_loop(0, _GROUPS, add_group, 0, unroll=8)

        pltpu.make_async_copy(obuf, o_hbm.at[pl.ds(x_off(b), _CH)], osem).start()

        @pl.when(b + 2 < B)
        def _():
            pltpu.make_async_copy(
                x_hbm.at[pl.ds(x_off(b + 2), _CH)], ibuf, isem).start()

    def loop(i, carry):
        b0 = i * 2
        step(b0, ib0, ob0, is0, os0)
        step(b0 + 1, ib1, ob1, is1, os1)
        return carry

    lax.fori_loop(0, B // 2, loop, 0)

    pltpu.make_async_copy(ob0, o_hbm.at[pl.ds(x_off(B - 2), _CH)], os0).wait()
    pltpu.make_async_copy(ob1, o_hbm.at[pl.ds(x_off(B - 1), _CH)], os1).wait()


def kernel(encoded_patches, pos_table):
    xflat = encoded_patches.reshape(B * P * D)
    tflat = pos_table.reshape(P * D)
    mesh = plsc.VectorSubcoreMesh(core_axis_name="c", subcore_axis_name="s")
    k = functools.partial(
        pl.kernel,
        mesh=mesh,
        out_type=jax.ShapeDtypeStruct((B * P * D,), jnp.float32),
        scratch_types=[
            pltpu.VMEM((_CH,), jnp.float32),
            pltpu.VMEM((_CH,), jnp.float32),
            pltpu.VMEM((_CH,), jnp.float32),
            pltpu.VMEM((_CH,), jnp.float32),
            pltpu.VMEM((_CH,), jnp.float32),
            pltpu.SemaphoreType.DMA,
            pltpu.SemaphoreType.DMA,
            pltpu.SemaphoreType.DMA,
            pltpu.SemaphoreType.DMA,
        ],
    )(_sc_body)
    out = k(xflat, tflat)
    return out.reshape(B, P, D)


# speedup vs baseline: 3.7852x; 3.7852x over previous
import functools

import jax
import jax.numpy as jnp
from jax import lax
from jax.experimental import pallas as pl
from jax.experimental.pallas import tpu as pltpu
from jax.experimental.pallas import tpu_sc as plsc

B = 64
P = 576
D = 768

_NC = 2
_NS = 16
_NW = _NC * _NS
_RPW = (B * P) // _NW
_CPB = 16
_NCH = _RPW // _CPB
_GPR = D // 16


def _sc_body(x_hbm, t_hbm, o_hbm, spos,
             pb0, pb1, ib0, ib1, ob0, ob1,
             ps0, ps1, is0, is1, os0, os1):
    cid = lax.axis_index("c")
    sid = lax.axis_index("s")
    wid = sid * _NC + cid
    row0 = wid * _RPW

    @pl.when(sid == 0)
    def _():
        pltpu.sync_copy(t_hbm, spos)

    plsc.subcore_barrier()

    def xrow(c):
        return row0 + c * _CPB

    def prow(c):
        return (c % (P // _CPB)) * _CPB

    def start_in(c, pbuf, ibuf, psem, isem):
        pltpu.make_async_copy(
            x_hbm.at[pl.ds(xrow(c), _CPB), :], ibuf, isem).start()
        pltpu.make_async_copy(
            spos.at[pl.ds(prow(c), _CPB), :], pbuf, psem).start()

    start_in(0, pb0, ib0, ps0, is0)
    start_in(1, pb1, ib1, ps1, is1)

    def step(c, pbuf, ibuf, obuf, psem, isem, osem):
        pltpu.make_async_copy(
            x_hbm.at[pl.ds(xrow(c), _CPB), :], ibuf, isem).wait()
        pltpu.make_async_copy(
            spos.at[pl.ds(prow(c), _CPB), :], pbuf, psem).wait()

        @pl.when(c >= 2)
        def _():
            pltpu.make_async_copy(
                obuf, o_hbm.at[pl.ds(xrow(c - 2), _CPB), :], osem).wait()

        @plsc.parallel_loop(0, _CPB * _GPR, 1, unroll=8)
        def _add_group(i):
            r = i // _GPR
            g = (i % _GPR) * 16
            obuf[r, pl.ds(g, 16)] = ibuf[r, pl.ds(g, 16)] + pbuf[r, pl.ds(g, 16)]

        pltpu.make_async_copy(
            obuf, o_hbm.at[pl.ds(xrow(c), _CPB), :], osem).start()

        @pl.when(c + 2 < _NCH)
        def _():
            start_in(c + 2, pbuf, ibuf, psem, isem)

    def loop(i, carry):
        c0 = i * 2
        step(c0, pb0, ib0, ob0, ps0, is0, os0)
        step(c0 + 1, pb1, ib1, ob1, ps1, is1, os1)
        return carry

    lax.fori_loop(0, _NCH // 2, loop, 0)

    pltpu.make_async_copy(
        ob0, o_hbm.at[pl.ds(xrow(_NCH - 2), _CPB), :], os0).wait()
    pltpu.make_async_copy(
        ob1, o_hbm.at[pl.ds(xrow(_NCH - 1), _CPB), :], os1).wait()


def kernel(encoded_patches, pos_table):
    x2 = encoded_patches.reshape(B * P, D)
    mesh = plsc.VectorSubcoreMesh(core_axis_name="c", subcore_axis_name="s")
    k = functools.partial(
        pl.kernel,
        mesh=mesh,
        out_type=jax.ShapeDtypeStruct((B * P, D), jnp.float32),
        scratch_types=[
            pltpu.VMEM_SHARED((P, D), jnp.float32),
            pltpu.VMEM((_CPB, D), jnp.float32),
            pltpu.VMEM((_CPB, D), jnp.float32),
            pltpu.VMEM((_CPB, D), jnp.float32),
            pltpu.VMEM((_CPB, D), jnp.float32),
            pltpu.VMEM((_CPB, D), jnp.float32),
            pltpu.VMEM((_CPB, D), jnp.float32),
            pltpu.SemaphoreType.DMA,
            pltpu.SemaphoreType.DMA,
            pltpu.SemaphoreType.DMA,
            pltpu.SemaphoreType.DMA,
            pltpu.SemaphoreType.DMA,
            pltpu.SemaphoreType.DMA,
        ],
    )(_sc_body)
    out = k(x2, pos_table)
    return out.reshape(B, P, D)


# speedup vs baseline: 3.7891x; 1.0010x over previous
import functools

import jax
import jax.numpy as jnp
from jax import lax
from jax.experimental import pallas as pl
from jax.experimental.pallas import tpu as pltpu
from jax.experimental.pallas import tpu_sc as plsc

B = 64
P = 576
D = 768

_NC = 2
_NS = 16
_NW = _NC * _NS
_RPW = (B * P) // _NW
_CPB = 16
_NCH = _RPW // _CPB
_GPR = D // 16


def _sc_body(x_hbm, t_hbm, o_hbm, spos,
             pb0, pb1, xb0, xb1, xb2, xb3,
             ps0, ps1, is0, is1, is2, is3, os0, os1, os2, os3):
    cid = lax.axis_index("c")
    sid = lax.axis_index("s")
    wid = sid * _NC + cid
    row0 = wid * _RPW

    @pl.when(sid == 0)
    def _():
        pltpu.sync_copy(t_hbm, spos)

    plsc.subcore_barrier()

    xbufs = (xb0, xb1, xb2, xb3)
    isems = (is0, is1, is2, is3)
    osems = (os0, os1, os2, os3)
    pbufs = (pb0, pb1)
    psems = (ps0, ps1)

    def xrow(c):
        return row0 + c * _CPB

    def prow(c):
        return (c % (P // _CPB)) * _CPB

    def start_in(c, sx, sp):
        pltpu.make_async_copy(
            x_hbm.at[pl.ds(xrow(c), _CPB), :], xbufs[sx], isems[sx]).start()
        pltpu.make_async_copy(
            spos.at[pl.ds(prow(c), _CPB), :], pbufs[sp], psems[sp]).start()

    start_in(0, 0, 0)
    start_in(1, 1, 1)

    def step(c, sx, sp):
        guard_wait = c >= 2
        guard_pref = c + 2 < _NCH
        xbuf, isem, osem = xbufs[sx], isems[sx], osems[sx]
        pbuf, psem = pbufs[sp], psems[sp]

        pltpu.make_async_copy(
            x_hbm.at[pl.ds(xrow(c), _CPB), :], xbuf, isem).wait()
        pltpu.make_async_copy(
            spos.at[pl.ds(prow(c), _CPB), :], pbuf, psem).wait()

        @plsc.parallel_loop(0, _CPB * _GPR, 1, unroll=8)
        def _add_group(i):
            r = i // _GPR
            g = (i % _GPR) * 16
            plsc.addupdate(xbuf.at[r, pl.ds(g, 16)], pbuf[r, pl.ds(g, 16)])

        pltpu.make_async_copy(
            xbuf, o_hbm.at[pl.ds(xrow(c), _CPB), :], osem).start()

        sx2 = (sx + 2) % 4

        @pl.when(guard_pref)
        def _():
            @pl.when(guard_wait)
            def _():
                pltpu.make_async_copy(
                    xbufs[sx2],
                    o_hbm.at[pl.ds(xrow(c - 2), _CPB), :],
                    osems[sx2]).wait()

            start_in(c + 2, sx2, sp)

    def loop(i, carry):
        c0 = i * 4
        step(c0, 0, 0)
        step(c0 + 1, 1, 1)
        step(c0 + 2, 2, 0)
        step(c0 + 3, 3, 1)
        return carry

    lax.fori_loop(0, _NCH // 4, loop, 0)

    for k in range(4):
        c = _NCH - 4 + k
        pltpu.make_async_copy(
            xbufs[c % 4], o_hbm.at[pl.ds(xrow(c), _CPB), :],
            osems[c % 4]).wait()


def kernel(encoded_patches, pos_table):
    x2 = encoded_patches.reshape(B * P, D)
    mesh = plsc.VectorSubcoreMesh(core_axis_name="c", subcore_axis_name="s")
    k = functools.partial(
        pl.kernel,
        mesh=mesh,
        out_type=jax.ShapeDtypeStruct((B * P, D), jnp.float32),
        scratch_types=[
            pltpu.VMEM_SHARED((P, D), jnp.float32),
            pltpu.VMEM((_CPB, D), jnp.float32),
            pltpu.VMEM((_CPB, D), jnp.float32),
            pltpu.VMEM((_CPB, D), jnp.float32),
            pltpu.VMEM((_CPB, D), jnp.float32),
            pltpu.VMEM((_CPB, D), jnp.float32),
            pltpu.VMEM((_CPB, D), jnp.float32),
            pltpu.SemaphoreType.DMA,
            pltpu.SemaphoreType.DMA,
            pltpu.SemaphoreType.DMA,
            pltpu.SemaphoreType.DMA,
            pltpu.SemaphoreType.DMA,
            pltpu.SemaphoreType.DMA,
            pltpu.SemaphoreType.DMA,
            pltpu.SemaphoreType.DMA,
            pltpu.SemaphoreType.DMA,
            pltpu.SemaphoreType.DMA,
        ],
    )(_sc_body)
    out = k(x2, pos_table)
    return out.reshape(B, P, D)
